# baseline (speedup 1.0000x reference)
import functools

import jax
import jax.numpy as jnp
from jax import lax
from jax.experimental import pallas as pl
from jax.experimental.pallas import tpu as pltpu
from jax.experimental.pallas import tpu_sc as plsc

CTX = 1000
B, S = 1024, 50
ROWS = B * S
NW = 32
RPW = ROWS // NW
CHUNK = 32
NCHUNK = RPW // CHUNK
CW = CHUNK * CTX
L = 16


def _body(seq_hbm, out_hbm, idx_v, buf0, buf1, sem0, sem1):
    cid = lax.axis_index("c")
    sid = lax.axis_index("s")
    wid = sid * 2 + cid
    row0 = wid * RPW

    pltpu.sync_copy(seq_hbm.at[pl.ds(row0, RPW)], idx_v)

    zero16 = jnp.zeros((L,), jnp.float32)
    one16 = jnp.full((L,), 1.0, jnp.float32)
    rowoff = lax.iota(jnp.int32, L) * CTX

    UNROLL = 25
    def _zero_body(i, carry):
        base = i * (UNROLL * L)
        for k in range(UNROLL):
            buf0[pl.ds(base + k * L, L)] = zero16
            buf1[pl.ds(base + k * L, L)] = zero16
        return carry
    lax.fori_loop(0, CW // (UNROLL * L), _zero_body, 0)

    bufs = (buf0, buf1)
    sems = (sem0, sem1)

    def _flat_idx(c, o):
        idxs = idx_v[pl.ds(c * CHUNK + o, L)]
        return idxs + (rowoff + o * CTX)

    handles = [None, None]
    pending = [None, None]
    for c in range(NCHUNK):
        bsel = c & 1
        buf = bufs[bsel]
        if handles[bsel] is not None:
            handles[bsel].wait()
            pc = pending[bsel]
            for o in range(0, CHUNK, L):
                plsc.store_scatter(buf, [_flat_idx(pc, o)], zero16)
        for o in range(0, CHUNK, L):
            plsc.store_scatter(buf, [_flat_idx(c, o)], one16)
        dst = out_hbm.at[pl.ds((row0 + c * CHUNK) * CTX, CW)]
        handles[bsel] = pltpu.async_copy(buf, dst, sems[bsel])
        pending[bsel] = c
    handles[0].wait()
    handles[1].wait()


@jax.jit
def _onehot_sc(seq_flat):
    kern = functools.partial(
        pl.kernel,
        mesh=plsc.VectorSubcoreMesh(core_axis_name="c", subcore_axis_name="s"),
        out_type=jax.ShapeDtypeStruct((ROWS * CTX,), jnp.float32),
        scratch_types=[
            pltpu.VMEM((RPW,), jnp.int32),
            pltpu.VMEM((CW,), jnp.float32),
            pltpu.VMEM((CW,), jnp.float32),
            pltpu.SemaphoreType.DMA,
            pltpu.SemaphoreType.DMA,
        ],
        compiler_params=pltpu.CompilerParams(needs_layout_passes=False),
    )(_body)
    return kern(seq_flat)


def kernel(sequence):
    seq_flat = sequence.reshape(ROWS).astype(jnp.int32)
    out = _onehot_sc(seq_flat)
    return out.reshape(B, S, CTX)


# speedup vs baseline: 1.0010x; 1.0010x over previous
import functools

import jax
import jax.numpy as jnp
from jax import lax
from jax.experimental import pallas as pl
from jax.experimental.pallas import tpu as pltpu
from jax.experimental.pallas import tpu_sc as plsc

CTX = 1000
B, S = 1024, 50
ROWS = B * S
NW = 32
RPW = ROWS // NW
CHUNK = 64
NCHUNK = RPW // CHUNK
CW = CHUNK * CTX
L = 16


def _body(seq_hbm, out_hbm, idx_v, buf0, buf1, sem0, sem1):
    cid = lax.axis_index("c")
    sid = lax.axis_index("s")
    wid = sid * 2 + cid
    row0 = wid * RPW

    pltpu.sync_copy(seq_hbm.at[pl.ds(row0, RPW)], idx_v)

    zero16 = jnp.zeros((L,), jnp.float32)
    one16 = jnp.full((L,), 1.0, jnp.float32)
    rowoff = lax.iota(jnp.int32, L) * CTX

    UNROLL = 25
    def _zero_body(i, carry):
        base = i * (UNROLL * L)
        for k in range(UNROLL):
            buf0[pl.ds(base + k * L, L)] = zero16
            buf1[pl.ds(base + k * L, L)] = zero16
        return carry
    lax.fori_loop(0, CW // (UNROLL * L), _zero_body, 0)

    bufs = (buf0, buf1)
    sems = (sem0, sem1)

    def _flat_idx(c, o):
        idxs = idx_v[pl.ds(c * CHUNK + o, L)]
        return idxs + (rowoff + o * CTX)

    handles = [None, None]
    pending = [None, None]
    for c in range(NCHUNK):
        bsel = c & 1
        buf = bufs[bsel]
        if handles[bsel] is not None:
            handles[bsel].wait()
            pc = pending[bsel]
            for o in range(0, CHUNK, L):
                plsc.store_scatter(buf, [_flat_idx(pc, o)], zero16)
        for o in range(0, CHUNK, L):
            plsc.store_scatter(buf, [_flat_idx(c, o)], one16)
        dst = out_hbm.at[pl.ds((row0 + c * CHUNK) * CTX, CW)]
        handles[bsel] = pltpu.async_copy(buf, dst, sems[bsel])
        pending[bsel] = c
    handles[0].wait()
    handles[1].wait()


@jax.jit
def _onehot_sc(seq_flat):
    kern = functools.partial(
        pl.kernel,
        mesh=plsc.VectorSubcoreMesh(core_axis_name="c", subcore_axis_name="s"),
        out_type=jax.ShapeDtypeStruct((ROWS * CTX,), jnp.float32),
        scratch_types=[
            pltpu.VMEM((RPW,), jnp.int32),
            pltpu.VMEM((CW,), jnp.float32),
            pltpu.VMEM((CW,), jnp.float32),
            pltpu.SemaphoreType.DMA,
            pltpu.SemaphoreType.DMA,
        ],
        compiler_params=pltpu.CompilerParams(needs_layout_passes=False),
    )(_body)
    return kern(seq_flat)


def kernel(sequence):
    seq_flat = sequence.reshape(ROWS).astype(jnp.int32)
    out = _onehot_sc(seq_flat)
    return out.reshape(B, S, CTX)
